# Initial kernel scaffold; baseline (speedup 1.0000x reference)
import functools

import jax
import jax.numpy as jnp
from jax import lax
from jax.experimental import pallas as pl
from jax.experimental.pallas import tpu as pltpu
from jax.experimental.pallas import tpu_sc as plsc

_N = 10000
_E = 320000
_D = 128

_NC = 2
_NS = 16
_NW = _NC * _NS
_CHUNK = 128
_CPW = -(-_E // (_NW * _CHUNK))
_E_PAD = _NW * _CPW * _CHUNK
_NP = 10240
_RPT = _NP // _NS
_CW = 16

_mesh = plsc.VectorSubcoreMesh(
    core_axis_name="c", subcore_axis_name="s", num_cores=_NC, num_subcores=_NS
)


def _agg_body(with_counts, x_hbm, src_hbm, dst_hbm, zrow_hbm, zcnt_hbm, ones_hbm,
              sum_out, cnt_out, src_v, dst_v, rows_v, ones_v, acc_sh, cnt_sh, sem):
    c = lax.axis_index("c")
    s = lax.axis_index("s")
    wid = c * _NS + s
    r0 = s * _RPT
    pltpu.sync_copy(zrow_hbm, acc_sh.at[pl.ds(r0, _RPT)])
    if with_counts:
        pltpu.sync_copy(zcnt_hbm, cnt_sh.at[pl.ds(r0, _RPT)])
        pltpu.sync_copy(ones_hbm, ones_v)
    pltpu.sync_copy(src_hbm.at[wid], src_v)
    pltpu.sync_copy(dst_hbm.at[wid], dst_v)
    plsc.subcore_barrier()

    def body(j, carry):
        pltpu.async_copy(x_hbm.at[src_v.at[j]], rows_v, sem).wait()
        pltpu.sync_copy(rows_v, acc_sh.at[dst_v.at[j]], add=True)
        if with_counts:
            pltpu.sync_copy(ones_v, cnt_sh.at[dst_v.at[j]], add=True)
        return carry

    lax.fori_loop(0, _CPW, body, 0)
    plsc.subcore_barrier()
    pltpu.sync_copy(acc_sh.at[pl.ds(r0, _RPT)], sum_out.at[c, pl.ds(r0, _RPT)])
    if with_counts:
        pltpu.sync_copy(cnt_sh.at[pl.ds(r0, _RPT)], cnt_out.at[c, pl.ds(r0, _RPT)])


def _make_agg(with_counts):
    out_type = [jax.ShapeDtypeStruct((_NC, _NP, _D), jnp.float32)]
    if with_counts:
        out_type.append(jax.ShapeDtypeStruct((_NC, _NP, _CW), jnp.float32))
    return pl.kernel(
        functools.partial(_agg_body, with_counts),
        out_type=tuple(out_type) if with_counts else out_type[0],
        mesh=_mesh,
        scratch_types=[
            pltpu.VMEM((_CPW, _CHUNK), jnp.int32),
            pltpu.VMEM((_CPW, _CHUNK), jnp.int32),
            pltpu.VMEM((_CHUNK, _D), jnp.float32),
            pltpu.VMEM((_CHUNK, _CW), jnp.float32),
            pltpu.VMEM_SHARED((_NP, _D), jnp.float32),
            pltpu.VMEM_SHARED((_NP, _CW), jnp.float32),
            pltpu.SemaphoreType.DMA,
        ],
        name="sage_agg_cnt" if with_counts else "sage_agg",
    )


_agg_with_counts = _make_agg(True)
_agg_no_counts = _make_agg(False)

_BLK = 512
_GRID = _NP // _BLK


def _sage_tc1_body(p_ref, c_ref, x_ref, wl_ref, bl_ref, wr_ref, o_ref):
    ssum = p_ref[0] + p_ref[1]
    cnt = c_ref[0, :, :1] + c_ref[1, :, :1]
    mean = ssum / jnp.maximum(cnt, 1.0)
    out = (
        jnp.dot(mean, wl_ref[...], preferred_element_type=jnp.float32)
        + bl_ref[...]
        + jnp.dot(x_ref[...], wr_ref[...], preferred_element_type=jnp.float32)
    )
    nrm = jnp.sqrt(jnp.sum(out * out, axis=-1, keepdims=True))
    out = out / jnp.maximum(nrm, 1e-12)
    o_ref[...] = jnp.maximum(out, 0.0)


def _sage_tc2_body(p_ref, c_ref, x_ref, wl_ref, bl_ref, wr_ref, wo_ref, bo_ref, o_ref):
    ssum = p_ref[0] + p_ref[1]
    cnt = c_ref[0, :, :1] + c_ref[1, :, :1]
    mean = ssum / jnp.maximum(cnt, 1.0)
    out = (
        jnp.dot(mean, wl_ref[...], preferred_element_type=jnp.float32)
        + bl_ref[...]
        + jnp.dot(x_ref[...], wr_ref[...], preferred_element_type=jnp.float32)
    )
    nrm = jnp.sqrt(jnp.sum(out * out, axis=-1, keepdims=True))
    z = jnp.maximum(out / jnp.maximum(nrm, 1e-12), 0.0)
    o_ref[...] = jnp.dot(z, wo_ref[...], preferred_element_type=jnp.float32) + bo_ref[...]


_w_spec = pl.BlockSpec((_D, _D), lambda i: (0, 0))
_b_spec = pl.BlockSpec((1, _D), lambda i: (0, 0))
_row_spec = pl.BlockSpec((_BLK, _D), lambda i: (i, 0))
_p_spec = pl.BlockSpec((_NC, _BLK, _D), lambda i: (0, i, 0))
_c_spec = pl.BlockSpec((_NC, _BLK, _CW), lambda i: (0, i, 0))

_sage_tc1 = pl.pallas_call(
    _sage_tc1_body,
    grid=(_GRID,),
    in_specs=[_p_spec, _c_spec, _row_spec, _w_spec, _b_spec, _w_spec],
    out_specs=_row_spec,
    out_shape=jax.ShapeDtypeStruct((_NP, _D), jnp.float32),
)

_sage_tc2 = pl.pallas_call(
    _sage_tc2_body,
    grid=(_GRID,),
    in_specs=[_p_spec, _c_spec, _row_spec, _w_spec, _b_spec, _w_spec, _w_spec, _b_spec],
    out_specs=_row_spec,
    out_shape=jax.ShapeDtypeStruct((_NP, _D), jnp.float32),
)


@jax.jit
def kernel(x, edge_index, W1l, b1l, W1r, W2l, b2l, W2r, Wlin, blin):
    src = edge_index[0].astype(jnp.int32)
    dst = edge_index[1].astype(jnp.int32)
    pad = _E_PAD - _E
    src_p = jnp.concatenate([src, jnp.zeros((pad,), jnp.int32)]).reshape(_NW, _CPW, _CHUNK)
    dst_p = jnp.concatenate([dst, jnp.full((pad,), _N, jnp.int32)]).reshape(_NW, _CPW, _CHUNK)
    x_p = jnp.zeros((_NP, _D), jnp.float32).at[:_N].set(x)
    zrow = jnp.zeros((_RPT, _D), jnp.float32)
    zcnt = jnp.zeros((_RPT, _CW), jnp.float32)
    ones = jnp.ones((_CHUNK, _CW), jnp.float32)

    sums1, cnts = _agg_with_counts(x_p, src_p, dst_p, zrow, zcnt, ones)
    h1 = _sage_tc1(sums1, cnts, x_p, W1l.T, b1l.reshape(1, _D), W1r.T)
    sums2 = _agg_no_counts(h1, src_p, dst_p, zrow, zcnt, ones)
    out = _sage_tc2(
        sums2, cnts, h1, W2l.T, b2l.reshape(1, _D), W2r.T, Wlin.T, blin.reshape(1, _D)
    )
    return out[:_N]


# speedup vs baseline: 4.5894x; 4.5894x over previous
import functools

import jax
import jax.numpy as jnp
from jax import lax
from jax.experimental import pallas as pl
from jax.experimental.pallas import tpu as pltpu
from jax.experimental.pallas import tpu_sc as plsc

_N = 10000
_E = 320000
_D = 128

_NC = 2
_NS = 16
_NW = _NC * _NS
_CHUNK = 128
_CPW = -(-_E // (_NW * _CHUNK))
_E_PAD = _NW * _CPW * _CHUNK
_NP = 10240
_RPT = _NP // _NS
_CW = 128

_mesh = plsc.VectorSubcoreMesh(
    core_axis_name="c", subcore_axis_name="s", num_cores=_NC, num_subcores=_NS
)


def _cnt_body(dst_hbm, zcnt_hbm, ones_hbm,
              cnt_out, dst_v, ones_v, cnt_sh, sem):
    c = lax.axis_index("c")
    s = lax.axis_index("s")
    wid = c * _NS + s
    r0 = s * _RPT
    pltpu.sync_copy(zcnt_hbm, cnt_sh.at[pl.ds(r0, _RPT)])
    pltpu.sync_copy(ones_hbm, ones_v)
    pltpu.sync_copy(dst_hbm.at[wid], dst_v)
    plsc.subcore_barrier()

    def body(j, carry):
        pltpu.sync_copy(ones_v, cnt_sh.at[dst_v.at[j]], add=True)
        return carry

    lax.fori_loop(0, _CPW, body, 0)
    plsc.subcore_barrier()
    pltpu.sync_copy(cnt_sh.at[pl.ds(r0, _RPT)], cnt_out.at[c, pl.ds(r0, _RPT)])


def _agg_body(x_hbm, src_hbm, dst_hbm, zrow_hbm,
              sum_out, src_v, dst_v, rows_v, acc_sh, sem):
    c = lax.axis_index("c")
    s = lax.axis_index("s")
    wid = c * _NS + s
    r0 = s * _RPT
    pltpu.sync_copy(zrow_hbm, acc_sh.at[pl.ds(r0, _RPT)])
    pltpu.sync_copy(src_hbm.at[wid], src_v)
    pltpu.sync_copy(dst_hbm.at[wid], dst_v)
    plsc.subcore_barrier()

    def body(j, carry):
        pltpu.async_copy(x_hbm.at[src_v.at[j]], rows_v, sem).wait()
        pltpu.sync_copy(rows_v, acc_sh.at[dst_v.at[j]], add=True)
        return carry

    lax.fori_loop(0, _CPW, body, 0)
    plsc.subcore_barrier()
    pltpu.sync_copy(acc_sh.at[pl.ds(r0, _RPT)], sum_out.at[c, pl.ds(r0, _RPT)])


_agg = pl.kernel(
    _agg_body,
    out_type=jax.ShapeDtypeStruct((_NC, _NP, _D), jnp.float32),
    mesh=_mesh,
    scratch_types=[
        pltpu.VMEM((_CPW, _CHUNK), jnp.int32),
        pltpu.VMEM((_CPW, _CHUNK), jnp.int32),
        pltpu.VMEM((_CHUNK, _D), jnp.float32),
        pltpu.VMEM_SHARED((_NP, _D), jnp.float32),
        pltpu.SemaphoreType.DMA,
    ],
    name="sage_agg",
)

_cnt = pl.kernel(
    _cnt_body,
    out_type=jax.ShapeDtypeStruct((_NC, _NP, _CW), jnp.float32),
    mesh=_mesh,
    scratch_types=[
        pltpu.VMEM((_CPW, _CHUNK), jnp.int32),
        pltpu.VMEM((_CHUNK, _CW), jnp.float32),
        pltpu.VMEM_SHARED((_NP, _CW), jnp.float32),
        pltpu.SemaphoreType.DMA,
    ],
    name="sage_cnt",
)

_BLK = 512
_GRID = _NP // _BLK


def _sage_tc1_body(p_ref, c_ref, x_ref, wl_ref, bl_ref, wr_ref, o_ref):
    ssum = p_ref[0] + p_ref[1]
    cnt = c_ref[0, :, :1] + c_ref[1, :, :1]
    mean = ssum / jnp.maximum(cnt, 1.0)
    out = (
        jnp.dot(mean, wl_ref[...], preferred_element_type=jnp.float32)
        + bl_ref[...]
        + jnp.dot(x_ref[...], wr_ref[...], preferred_element_type=jnp.float32)
    )
    nrm = jnp.sqrt(jnp.sum(out * out, axis=-1, keepdims=True))
    out = out / jnp.maximum(nrm, 1e-12)
    o_ref[...] = jnp.maximum(out, 0.0)


def _sage_tc2_body(p_ref, c_ref, x_ref, wl_ref, bl_ref, wr_ref, wo_ref, bo_ref, o_ref):
    ssum = p_ref[0] + p_ref[1]
    cnt = c_ref[0, :, :1] + c_ref[1, :, :1]
    mean = ssum / jnp.maximum(cnt, 1.0)
    out = (
        jnp.dot(mean, wl_ref[...], preferred_element_type=jnp.float32)
        + bl_ref[...]
        + jnp.dot(x_ref[...], wr_ref[...], preferred_element_type=jnp.float32)
    )
    nrm = jnp.sqrt(jnp.sum(out * out, axis=-1, keepdims=True))
    z = jnp.maximum(out / jnp.maximum(nrm, 1e-12), 0.0)
    o_ref[...] = jnp.dot(z, wo_ref[...], preferred_element_type=jnp.float32) + bo_ref[...]


_w_spec = pl.BlockSpec((_D, _D), lambda i: (0, 0))
_b_spec = pl.BlockSpec((1, _D), lambda i: (0, 0))
_row_spec = pl.BlockSpec((_BLK, _D), lambda i: (i, 0))
_p_spec = pl.BlockSpec((_NC, _BLK, _D), lambda i: (0, i, 0))
_c_spec = pl.BlockSpec((_NC, _BLK, _CW), lambda i: (0, i, 0))

_sage_tc1 = pl.pallas_call(
    _sage_tc1_body,
    grid=(_GRID,),
    in_specs=[_p_spec, _c_spec, _row_spec, _w_spec, _b_spec, _w_spec],
    out_specs=_row_spec,
    out_shape=jax.ShapeDtypeStruct((_NP, _D), jnp.float32),
)

_sage_tc2 = pl.pallas_call(
    _sage_tc2_body,
    grid=(_GRID,),
    in_specs=[_p_spec, _c_spec, _row_spec, _w_spec, _b_spec, _w_spec, _w_spec, _b_spec],
    out_specs=_row_spec,
    out_shape=jax.ShapeDtypeStruct((_NP, _D), jnp.float32),
)


@jax.jit
def kernel(x, edge_index, W1l, b1l, W1r, W2l, b2l, W2r, Wlin, blin):
    src = edge_index[0].astype(jnp.int32)
    dst = edge_index[1].astype(jnp.int32)
    pad = _E_PAD - _E
    src_p = jnp.concatenate([src, jnp.zeros((pad,), jnp.int32)]).reshape(_NW, _CPW, _CHUNK)
    dst_p = jnp.concatenate([dst, jnp.full((pad,), _N, jnp.int32)]).reshape(_NW, _CPW, _CHUNK)
    x_p = jnp.zeros((_NP, _D), jnp.float32).at[:_N].set(x)
    zrow = jnp.zeros((_RPT, _D), jnp.float32)
    zcnt = jnp.zeros((_RPT, _CW), jnp.float32)
    ones = jnp.ones((_CHUNK, _CW), jnp.float32)

    sums1 = _agg(x_p, src_p, dst_p, zrow)
    cnts = _cnt(dst_p, zcnt, ones)
    h1 = _sage_tc1(sums1, cnts, x_p, W1l.T, b1l.reshape(1, _D), W1r.T)
    sums2 = _agg(h1, src_p, dst_p, zrow)
    out = _sage_tc2(
        sums2, cnts, h1, W2l.T, b2l.reshape(1, _D), W2r.T, Wlin.T, blin.reshape(1, _D)
    )
    return out[:_N]


# speedup vs baseline: 7.4275x; 1.6184x over previous
import jax
import jax.numpy as jnp
from jax import lax
from jax.experimental import pallas as pl
from jax.experimental.pallas import tpu as pltpu
from jax.experimental.pallas import tpu_sc as plsc

_N = 10000
_E = 320000
_D = 128

_NC = 2
_NS = 16
_NW = _NC * _NS
_CHUNK = 128
_C0 = 80
_C1 = 80
_NCH = _NS * (_C0 + _C1)
_CPW = _NCH // _NW
_E_PAD = _NCH * _CHUNK
_NP = 10240
_RPT = _NP // _NS
_CW = 128

_mesh = plsc.VectorSubcoreMesh(
    core_axis_name="c", subcore_axis_name="s", num_cores=_NC, num_subcores=_NS
)


def _cnt_body(dst_hbm, zcnt_hbm, ones_hbm,
              cnt_out, dst_v, ones_v, cnt_sh, sem):
    c = lax.axis_index("c")
    s = lax.axis_index("s")
    wid = c * _NS + s
    r0 = s * _RPT
    pltpu.sync_copy(zcnt_hbm, cnt_sh.at[pl.ds(r0, _RPT)])
    pltpu.sync_copy(ones_hbm, ones_v)
    pltpu.sync_copy(dst_hbm.at[wid], dst_v)
    plsc.subcore_barrier()

    def body(j, carry):
        pltpu.sync_copy(ones_v, cnt_sh.at[dst_v.at[j]], add=True)
        return carry

    lax.fori_loop(0, _CPW, body, 0)
    plsc.subcore_barrier()
    pltpu.sync_copy(cnt_sh.at[pl.ds(r0, _RPT)], cnt_out.at[c, pl.ds(r0, _RPT)])


def _agg_body(x_hbm, src_hbm, dst_hbm, zrow_hbm,
              sum_out, src_v, dst_v, rows_v, acc_sh, sem):
    c = lax.axis_index("c")
    s = lax.axis_index("s")
    r0 = s * _RPT
    nch = jnp.where(c == 0, _C0, _C1)
    base = jnp.where(c == 0, s * _C0, _NS * _C0 + s * _C1)
    sbase = pl.multiple_of(jnp.minimum(base, _NCH - _C0), 8)
    off = base - sbase
    with jax.named_scope("agg_init"):
        pltpu.sync_copy(zrow_hbm, acc_sh.at[pl.ds(r0, _RPT)])
        pltpu.sync_copy(src_hbm.at[pl.ds(sbase, _C0)], src_v)
        pltpu.sync_copy(dst_hbm.at[pl.ds(sbase, _C0)], dst_v)
        plsc.subcore_barrier()

    def body(j, carry):
        pltpu.async_copy(x_hbm.at[src_v.at[off + j]], rows_v, sem).wait()
        pltpu.sync_copy(rows_v, acc_sh.at[dst_v.at[off + j]], add=True)
        return carry

    with jax.named_scope("agg_loop"):
        lax.fori_loop(0, nch, body, 0)
    with jax.named_scope("agg_bar"):
        plsc.subcore_barrier()
    with jax.named_scope("agg_out"):
        pltpu.sync_copy(acc_sh.at[pl.ds(r0, _RPT)], sum_out.at[c, pl.ds(r0, _RPT)])


_agg = pl.kernel(
    _agg_body,
    out_type=jax.ShapeDtypeStruct((_NC, _NP, _D), jnp.float32),
    mesh=_mesh,
    scratch_types=[
        pltpu.VMEM((_C0, _CHUNK), jnp.int32),
        pltpu.VMEM((_C0, _CHUNK), jnp.int32),
        pltpu.VMEM((_CHUNK, _D), jnp.float32),
        pltpu.VMEM_SHARED((_NP, _D), jnp.float32),
        pltpu.SemaphoreType.DMA,
    ],
    name="sage_agg",
)

_cnt = pl.kernel(
    _cnt_body,
    out_type=jax.ShapeDtypeStruct((_NC, _NP, _CW), jnp.float32),
    mesh=_mesh,
    scratch_types=[
        pltpu.VMEM((_CPW, _CHUNK), jnp.int32),
        pltpu.VMEM((_CHUNK, _CW), jnp.float32),
        pltpu.VMEM_SHARED((_NP, _CW), jnp.float32),
        pltpu.SemaphoreType.DMA,
    ],
    name="sage_cnt",
)

_BLK = 512
_GRID = _NP // _BLK


def _sage_tc1_body(p_ref, c_ref, x_ref, wl_ref, bl_ref, wr_ref, o_ref):
    ssum = p_ref[0] + p_ref[1]
    cnt = c_ref[0, :, :1] + c_ref[1, :, :1]
    mean = ssum / jnp.maximum(cnt, 1.0)
    out = (
        jnp.dot(mean, wl_ref[...], preferred_element_type=jnp.float32)
        + bl_ref[...]
        + jnp.dot(x_ref[...], wr_ref[...], preferred_element_type=jnp.float32)
    )
    nrm = jnp.sqrt(jnp.sum(out * out, axis=-1, keepdims=True))
    out = out / jnp.maximum(nrm, 1e-12)
    o_ref[...] = jnp.maximum(out, 0.0)


def _sage_tc2_body(p_ref, c_ref, x_ref, wl_ref, bl_ref, wr_ref, wo_ref, bo_ref, o_ref):
    ssum = p_ref[0] + p_ref[1]
    cnt = c_ref[0, :, :1] + c_ref[1, :, :1]
    mean = ssum / jnp.maximum(cnt, 1.0)
    out = (
        jnp.dot(mean, wl_ref[...], preferred_element_type=jnp.float32)
        + bl_ref[...]
        + jnp.dot(x_ref[...], wr_ref[...], preferred_element_type=jnp.float32)
    )
    nrm = jnp.sqrt(jnp.sum(out * out, axis=-1, keepdims=True))
    z = jnp.maximum(out / jnp.maximum(nrm, 1e-12), 0.0)
    o_ref[...] = jnp.dot(z, wo_ref[...], preferred_element_type=jnp.float32) + bo_ref[...]


_w_spec = pl.BlockSpec((_D, _D), lambda i: (0, 0))
_b_spec = pl.BlockSpec((1, _D), lambda i: (0, 0))
_row_spec = pl.BlockSpec((_BLK, _D), lambda i: (i, 0))
_p_spec = pl.BlockSpec((_NC, _BLK, _D), lambda i: (0, i, 0))
_c_spec = pl.BlockSpec((_NC, _BLK, 16), lambda i: (0, i, 0))

_sage_tc1 = pl.pallas_call(
    _sage_tc1_body,
    grid=(_GRID,),
    in_specs=[_p_spec, _c_spec, _row_spec, _w_spec, _b_spec, _w_spec],
    out_specs=_row_spec,
    out_shape=jax.ShapeDtypeStruct((_NP, _D), jnp.float32),
)

_sage_tc2 = pl.pallas_call(
    _sage_tc2_body,
    grid=(_GRID,),
    in_specs=[_p_spec, _c_spec, _row_spec, _w_spec, _b_spec, _w_spec, _w_spec, _b_spec],
    out_specs=_row_spec,
    out_shape=jax.ShapeDtypeStruct((_NP, _D), jnp.float32),
)


@jax.jit
def kernel(x, edge_index, W1l, b1l, W1r, W2l, b2l, W2r, Wlin, blin):
    src = edge_index[0].astype(jnp.int32)
    dst = edge_index[1].astype(jnp.int32)
    pad = _E_PAD - _E
    pad_rng = _N + (jnp.arange(pad, dtype=jnp.int32) % (_NP - _N))
    src_p = jnp.concatenate([src, pad_rng]).reshape(_NCH, _CHUNK)
    dst_p = jnp.concatenate([dst, pad_rng]).reshape(_NCH, _CHUNK)
    dst_cnt = dst_p.reshape(_NW, _CPW, _CHUNK)
    x_p = jnp.zeros((_NP, _D), jnp.float32).at[:_N].set(x)
    zrow = jnp.zeros((_RPT, _D), jnp.float32)
    zcnt = jnp.zeros((_RPT, _CW), jnp.float32)
    ones = jnp.ones((_CHUNK, _CW), jnp.float32)

    sums1 = _agg(x_p, src_p, dst_p, zrow)
    cnts = _cnt(dst_cnt, zcnt, ones)[:, :, :16]
    h1 = _sage_tc1(sums1, cnts, x_p, W1l.T, b1l.reshape(1, _D), W1r.T)
    sums2 = _agg(h1, src_p, dst_p, zrow)
    out = _sage_tc2(
        sums2, cnts, h1, W2l.T, b2l.reshape(1, _D), W2r.T, Wlin.T, blin.reshape(1, _D)
    )
    return out[:_N]
